# Initial kernel scaffold; baseline (speedup 1.0000x reference)
import functools

import jax
import jax.numpy as jnp
from jax import lax
from jax.experimental import pallas as pl
from jax.experimental.pallas import tpu as pltpu
from jax.experimental.pallas import tpu_sc as plsc

NC = 2
NS = 16
NW = NC * NS
CH = 128
IB = 17
F = 4


def _mesh():
    return plsc.VectorSubcoreMesh(
        core_axis_name="c", subcore_axis_name="s", num_cores=NC, num_subcores=NS
    )


def _deg_body(nblk, rpt, dst2, ones_hbm, zeros_hbm, out_hbm,
              acc, idx_b, ones_v, sem_s):
    c = lax.axis_index("c")
    s = lax.axis_index("s")
    wid = c * NS + s
    r0 = s * rpt
    pltpu.sync_copy(zeros_hbm.at[pl.ds(r0, rpt)], acc.at[pl.ds(r0, rpt)])
    pltpu.sync_copy(ones_hbm, ones_v)
    plsc.subcore_barrier()
    base = wid * nblk * IB

    def blk(b, carry):
        row0 = base + b * IB
        pltpu.sync_copy(dst2.at[pl.ds(row0, IB)], idx_b)
        cps = [
            pltpu.async_copy(ones_v, acc.at[idx_b.at[j]], sem_s, add=True)
            for j in range(IB)
        ]
        for cp in cps:
            cp.wait()
        return carry

    lax.fori_loop(0, nblk, blk, 0)
    plsc.subcore_barrier()
    pltpu.sync_copy(acc.at[pl.ds(r0, rpt)], out_hbm.at[c, pl.ds(r0, rpt)])


def _agg_body(nblk, rpt, table, src2, dst2, zeros_hbm, out_hbm,
              acc, sidx_b, didx_b, rows_b, sem_g, sem_s):
    c = lax.axis_index("c")
    s = lax.axis_index("s")
    wid = c * NS + s
    r0 = s * rpt
    pltpu.sync_copy(zeros_hbm.at[pl.ds(r0, rpt)], acc.at[pl.ds(r0, rpt)])
    plsc.subcore_barrier()
    base = wid * nblk * IB

    def blk(b, carry):
        row0 = base + b * IB
        pltpu.sync_copy(src2.at[pl.ds(row0, IB)], sidx_b)
        pltpu.sync_copy(dst2.at[pl.ds(row0, IB)], didx_b)
        gs = [
            pltpu.async_copy(table.at[sidx_b.at[j]], rows_b.at[j], sem_g)
            for j in range(IB)
        ]
        ss = []
        for j in range(IB):
            gs[j].wait()
            ss.append(
                pltpu.async_copy(rows_b.at[j], acc.at[didx_b.at[j]], sem_s, add=True)
            )
        for cp in ss:
            cp.wait()
        return carry

    lax.fori_loop(0, nblk, blk, 0)
    plsc.subcore_barrier()
    pltpu.sync_copy(acc.at[pl.ds(r0, rpt)], out_hbm.at[c, pl.ds(r0, rpt)])


def _prep_body(degp, x_ref, w_ref, out_dinv, out_g, out_gs):
    deg = degp[0, :, 0] + degp[1, :, 0] + 1.0
    dinv = lax.rsqrt(deg)[:, None]
    g = lax.dot_general(x_ref[...], w_ref[...], (((1,), (1,)), ((), ())),
                        preferred_element_type=jnp.float32)
    out_dinv[...] = dinv
    out_g[...] = g
    out_gs[...] = g * dinv


def _mid_body(accp, dinv_ref, g_ref, b_ref, w_ref, out_g2, out_g2s):
    dinv = dinv_ref[...]
    h = jnp.tanh((accp[0] + accp[1] + dinv * g_ref[...]) * dinv + b_ref[...])
    g2 = lax.dot_general(h, w_ref[...], (((1,), (1,)), ((), ())),
                         preferred_element_type=jnp.float32)
    out_g2[...] = g2
    out_g2s[...] = g2 * dinv


def _fin_body(accp, dinv_ref, g_ref, b_ref, wc_ref, bc_ref, out_o, out_h):
    dinv = dinv_ref[...]
    h = jnp.tanh((accp[0] + accp[1] + dinv * g_ref[...]) * dinv + b_ref[...])
    out_h[...] = h
    out_o[...] = lax.dot_general(h, wc_ref[...], (((1,), (1,)), ((), ())),
                                 preferred_element_type=jnp.float32) + bc_ref[...]


def kernel(x, edge_index, W1, b1, W2, b2, Wc, bc):
    n, f_in = x.shape
    e = edge_index.shape[1]
    hid = W1.shape[0]
    ncls = Wc.shape[0]

    ept = CH * IB
    nblk = -(-e // (NW * ept))
    e_pad = nblk * NW * ept
    pad = e_pad - e
    src_p = jnp.concatenate([edge_index[0], jnp.zeros((pad,), jnp.int32)])
    dst_p = jnp.concatenate([edge_index[1], jnp.full((pad,), n, jnp.int32)])
    src2 = src_p.reshape(-1, CH)
    dst2 = dst_p.reshape(-1, CH)

    rpt = -(-(n + 1) // NS)
    rpt += rpt % 2
    npad = NS * rpt
    zeros = jnp.zeros((npad, F), jnp.float32)
    ones = jnp.ones((CH, F), jnp.float32)

    mesh = _mesh()
    acc_t = jax.ShapeDtypeStruct((NC, npad, F), jnp.float32)

    deg_fn = pl.kernel(
        functools.partial(_deg_body, nblk, rpt),
        out_type=acc_t,
        mesh=mesh,
        scratch_types=[
            pltpu.VMEM_SHARED((npad, F), jnp.float32),
            pltpu.VMEM((IB, CH), jnp.int32),
            pltpu.VMEM((CH, F), jnp.float32),
            pltpu.SemaphoreType.DMA,
        ],
    )
    agg_fn = pl.kernel(
        functools.partial(_agg_body, nblk, rpt),
        out_type=acc_t,
        mesh=mesh,
        scratch_types=[
            pltpu.VMEM_SHARED((npad, F), jnp.float32),
            pltpu.VMEM((IB, CH), jnp.int32),
            pltpu.VMEM((IB, CH), jnp.int32),
            pltpu.VMEM((IB, CH, F), jnp.float32),
            pltpu.SemaphoreType.DMA,
            pltpu.SemaphoreType.DMA,
        ],
    )

    bn = 2000 if n % 2000 == 0 else n
    grid = n // bn
    acc_spec = pl.BlockSpec((NC, bn, F), lambda i: (0, i, 0))
    vec_spec = pl.BlockSpec((bn, F), lambda i: (i, 0))
    col_spec = pl.BlockSpec((bn, 1), lambda i: (i, 0))
    full = lambda shape: pl.BlockSpec(shape, lambda i: tuple(0 for _ in shape))

    prep_fn = pl.pallas_call(
        _prep_body,
        grid=(grid,),
        in_specs=[acc_spec, pl.BlockSpec((bn, f_in), lambda i: (i, 0)), full(W1.shape)],
        out_specs=[col_spec, vec_spec, vec_spec],
        out_shape=[
            jax.ShapeDtypeStruct((n, 1), jnp.float32),
            jax.ShapeDtypeStruct((n, F), jnp.float32),
            jax.ShapeDtypeStruct((n, F), jnp.float32),
        ],
    )
    mid_fn = pl.pallas_call(
        _mid_body,
        grid=(grid,),
        in_specs=[acc_spec, col_spec, vec_spec, full((1, F)), full(W2.shape)],
        out_specs=[vec_spec, vec_spec],
        out_shape=[
            jax.ShapeDtypeStruct((n, F), jnp.float32),
            jax.ShapeDtypeStruct((n, F), jnp.float32),
        ],
    )
    fin_fn = pl.pallas_call(
        _fin_body,
        grid=(grid,),
        in_specs=[acc_spec, col_spec, vec_spec, full((1, F)), full(Wc.shape),
                  full((1, F))],
        out_specs=[vec_spec, vec_spec],
        out_shape=[
            jax.ShapeDtypeStruct((n, F), jnp.float32),
            jax.ShapeDtypeStruct((n, F), jnp.float32),
        ],
    )

    degp = deg_fn(dst2, ones, zeros)
    dinv, g1, g1s = prep_fn(degp, x, W1)
    acc1 = agg_fn(g1s, src2, dst2, zeros)
    g2, g2s = mid_fn(acc1, dinv, g1, b1.reshape(1, F), W2)
    acc2 = agg_fn(g2s, src2, dst2, zeros)
    out, h2 = fin_fn(acc2, dinv, g2, b2.reshape(1, F), Wc, bc.reshape(1, F))
    return out, h2


# speedup vs baseline: 38.0051x; 38.0051x over previous
import functools

import jax
import jax.numpy as jnp
from jax import lax
from jax.experimental import pallas as pl
from jax.experimental.pallas import tpu as pltpu
from jax.experimental.pallas import tpu_sc as plsc

NC = 2
NS = 16
NW = NC * NS
CH = 128
IB = 17
F = 4


def _mesh():
    return plsc.VectorSubcoreMesh(
        core_axis_name="c", subcore_axis_name="s", num_cores=NC, num_subcores=NS
    )


def _deg_body(nblk, rpt, dst2, ones_hbm, zeros_hbm, out_hbm,
              acc, idx_b, ones_v, sem_s):
    c = lax.axis_index("c")
    s = lax.axis_index("s")
    wid = c * NS + s
    r0 = s * rpt
    pltpu.sync_copy(zeros_hbm.at[pl.ds(r0, rpt)], acc.at[pl.ds(r0, rpt)])
    pltpu.sync_copy(ones_hbm, ones_v)
    plsc.subcore_barrier()
    base = wid * nblk

    def blk(b, carry):
        pltpu.sync_copy(dst2.at[base + b], idx_b)
        cps = [
            pltpu.async_copy(ones_v, acc.at[idx_b.at[j]], sem_s, add=True)
            for j in range(IB)
        ]
        for cp in cps:
            cp.wait()
        return carry

    lax.fori_loop(0, nblk, blk, 0)
    plsc.subcore_barrier()
    pltpu.sync_copy(acc.at[pl.ds(r0, rpt)], out_hbm.at[c, pl.ds(r0, rpt)])


def _agg_body(nblk, rpt, table, src2, dst2, zeros_hbm, out_hbm,
              acc, sidx_b, didx_b, rows_b, sem_g, sem_s):
    c = lax.axis_index("c")
    s = lax.axis_index("s")
    wid = c * NS + s
    r0 = s * rpt
    pltpu.sync_copy(zeros_hbm.at[pl.ds(r0, rpt)], acc.at[pl.ds(r0, rpt)])
    plsc.subcore_barrier()
    base = wid * nblk

    def blk(b, carry):
        pltpu.sync_copy(src2.at[base + b], sidx_b)
        pltpu.sync_copy(dst2.at[base + b], didx_b)
        gs = [
            pltpu.async_copy(table.at[sidx_b.at[j]], rows_b.at[j], sem_g)
            for j in range(IB)
        ]
        ss = []
        for j in range(IB):
            gs[j].wait()
            ss.append(
                pltpu.async_copy(rows_b.at[j], acc.at[didx_b.at[j]], sem_s, add=True)
            )
        for cp in ss:
            cp.wait()
        return carry

    lax.fori_loop(0, nblk, blk, 0)
    plsc.subcore_barrier()
    pltpu.sync_copy(acc.at[pl.ds(r0, rpt)], out_hbm.at[c, pl.ds(r0, rpt)])


def _prep_body(degp, x_ref, w_ref, out_dinv, out_g, out_gs):
    deg = degp[0, :, 0] + degp[1, :, 0] + 1.0
    dinv = lax.rsqrt(deg)[:, None]
    g = lax.dot_general(x_ref[...], w_ref[...], (((1,), (1,)), ((), ())),
                        preferred_element_type=jnp.float32)
    out_dinv[...] = dinv
    out_g[...] = g
    out_gs[...] = g * dinv


def _mid_body(accp, dinv_ref, g_ref, b_ref, w_ref, out_g2, out_g2s):
    dinv = dinv_ref[...]
    h = jnp.tanh((accp[0] + accp[1] + dinv * g_ref[...]) * dinv + b_ref[...])
    g2 = lax.dot_general(h, w_ref[...], (((1,), (1,)), ((), ())),
                         preferred_element_type=jnp.float32)
    out_g2[...] = g2
    out_g2s[...] = g2 * dinv


def _fin_body(accp, dinv_ref, g_ref, b_ref, wc_ref, bc_ref, out_o, out_h):
    dinv = dinv_ref[...]
    h = jnp.tanh((accp[0] + accp[1] + dinv * g_ref[...]) * dinv + b_ref[...])
    out_h[...] = h
    out_o[...] = lax.dot_general(h, wc_ref[...], (((1,), (1,)), ((), ())),
                                 preferred_element_type=jnp.float32) + bc_ref[...]


def kernel(x, edge_index, W1, b1, W2, b2, Wc, bc):
    n, f_in = x.shape
    e = edge_index.shape[1]
    hid = W1.shape[0]
    ncls = Wc.shape[0]

    ept = CH * IB
    nblk = -(-e // (NW * ept))
    e_pad = nblk * NW * ept
    pad = e_pad - e
    src_p = jnp.concatenate([edge_index[0], jnp.zeros((pad,), jnp.int32)])
    dst_p = jnp.concatenate([edge_index[1], jnp.full((pad,), n, jnp.int32)])
    src2 = src_p.reshape(-1, IB, CH)
    dst2 = dst_p.reshape(-1, IB, CH)

    rpt = -(-(n + 1) // NS)
    rpt = -(-rpt // 8) * 8
    npad = NS * rpt
    zeros = jnp.zeros((npad, F), jnp.float32)
    ones = jnp.ones((CH, F), jnp.float32)

    mesh = _mesh()
    acc_t = jax.ShapeDtypeStruct((NC, npad, F), jnp.float32)
    sc_params = pltpu.CompilerParams(use_tc_tiling_on_sc=False)

    deg_fn = pl.kernel(
        functools.partial(_deg_body, nblk, rpt),
        out_type=acc_t,
        mesh=mesh,
        compiler_params=sc_params,
        scratch_types=[
            pltpu.VMEM_SHARED((npad, F), jnp.float32),
            pltpu.VMEM((IB, CH), jnp.int32),
            pltpu.VMEM((CH, F), jnp.float32),
            pltpu.SemaphoreType.DMA,
        ],
    )
    agg_fn = pl.kernel(
        functools.partial(_agg_body, nblk, rpt),
        out_type=acc_t,
        mesh=mesh,
        compiler_params=sc_params,
        scratch_types=[
            pltpu.VMEM_SHARED((npad, F), jnp.float32),
            pltpu.VMEM((IB, CH), jnp.int32),
            pltpu.VMEM((IB, CH), jnp.int32),
            pltpu.VMEM((IB, CH, F), jnp.float32),
            pltpu.SemaphoreType.DMA,
            pltpu.SemaphoreType.DMA,
        ],
    )

    bn = 2000 if n % 2000 == 0 else n
    grid = n // bn
    acc_spec = pl.BlockSpec((NC, bn, F), lambda i: (0, i, 0))
    vec_spec = pl.BlockSpec((bn, F), lambda i: (i, 0))
    col_spec = pl.BlockSpec((bn, 1), lambda i: (i, 0))
    full = lambda shape: pl.BlockSpec(shape, lambda i: tuple(0 for _ in shape))

    prep_fn = pl.pallas_call(
        _prep_body,
        grid=(grid,),
        in_specs=[acc_spec, pl.BlockSpec((bn, f_in), lambda i: (i, 0)), full(W1.shape)],
        out_specs=[col_spec, vec_spec, vec_spec],
        out_shape=[
            jax.ShapeDtypeStruct((n, 1), jnp.float32),
            jax.ShapeDtypeStruct((n, F), jnp.float32),
            jax.ShapeDtypeStruct((n, F), jnp.float32),
        ],
    )
    mid_fn = pl.pallas_call(
        _mid_body,
        grid=(grid,),
        in_specs=[acc_spec, col_spec, vec_spec, full((1, F)), full(W2.shape)],
        out_specs=[vec_spec, vec_spec],
        out_shape=[
            jax.ShapeDtypeStruct((n, F), jnp.float32),
            jax.ShapeDtypeStruct((n, F), jnp.float32),
        ],
    )
    fin_fn = pl.pallas_call(
        _fin_body,
        grid=(grid,),
        in_specs=[acc_spec, col_spec, vec_spec, full((1, F)), full(Wc.shape),
                  full((1, F))],
        out_specs=[vec_spec, vec_spec],
        out_shape=[
            jax.ShapeDtypeStruct((n, F), jnp.float32),
            jax.ShapeDtypeStruct((n, F), jnp.float32),
        ],
    )

    degp = deg_fn(dst2, ones, zeros)
    dinv, g1, g1s = prep_fn(degp, x, W1)
    acc1 = agg_fn(g1s, src2, dst2, zeros)
    g2, g2s = mid_fn(acc1, dinv, g1, b1.reshape(1, F), W2)
    acc2 = agg_fn(g2s, src2, dst2, zeros)
    out, h2 = fin_fn(acc2, dinv, g2, b2.reshape(1, F), Wc, bc.reshape(1, F))
    return out, h2
